# Initial kernel scaffold; baseline (speedup 1.0000x reference)
import jax
import jax.numpy as jnp
from jax.experimental import pallas as pl


def kernel(z, y, log_cov, prior_logits):
    raise NotImplementedError("write your pallas kernel here")


# speedup vs baseline: 49.2405x; 49.2405x over previous
import math

import numpy as np
import jax
import jax.numpy as jnp
from jax.experimental import pallas as pl
from jax.experimental.pallas import tpu as pltpu

_C = 1000
_D = 2048
_CPAD = 1024
_N = 16384
_BLK = 1024
_GRID = _N // _BLK
_FISHER_EPS = 1e-08
_PRIOR_STRENGTH = 0.5


def _build_mu() -> np.ndarray:
    k = np.arange(1, _C, dtype=np.float64)
    inv = 1.0 / np.sqrt(k * (k + 1.0))
    r = np.arange(_C, dtype=np.float64)[:, None]
    kk = k[None, :]
    basis = np.where(r < kk, inv, np.where(r == kk, -kk * inv, 0.0))
    pairwise = math.sqrt(2.0 * _C / (_C - 1))
    scale = 6.0 / pairwise
    mu = basis * (scale / math.sqrt(1.0 - 1.0 / _C))
    out = np.zeros((_CPAD, _D), dtype=np.float32)
    out[:_C, : _C - 1] = mu.astype(np.float32)
    return out


_MU_F32 = _build_mu()
_MU_BF16 = _MU_F32.astype(jnp.bfloat16)
_M2 = (
    _MU_F32.astype(np.float64) ** 2
).sum(axis=1).astype(np.float32).reshape(1, _CPAD)


def _fisher_kernel(z_ref, y_ref, mu_ref, m2_ref, plp_ref, lcv_ref,
                   out_ref, acc_ref, counts_ref):
    i = pl.program_id(0)

    @pl.when(i == 0)
    def _init():
        acc_ref[0] = 0.0
        acc_ref[1] = 0.0
        counts_ref[...] = jnp.zeros_like(counts_ref)

    z = z_ref[...]
    y = y_ref[0, 0, :]
    yc = y.reshape(1, _BLK).T
    col = jax.lax.broadcasted_iota(jnp.int32, (_BLK, _CPAD), 1)
    onehot = (yc == col).astype(jnp.bfloat16)

    counts_ref[...] += jnp.sum(
        (yc == col).astype(jnp.float32), axis=0, keepdims=True)

    g = jax.lax.dot_general(
        onehot, mu_ref[...],
        (((1,), (0,)), ((), ())),
        preferred_element_type=jnp.float32)

    acc_ref[0] += jnp.sum(z * z)
    acc_ref[1] += jnp.sum(z * g)

    @pl.when(i == _GRID - 1)
    def _epilogue():
        counts = counts_ref[...]
        m2 = m2_ref[...]
        total = jnp.maximum(jnp.sum(counts), 1.0)

        lp = plp_ref[...]
        e = jnp.exp(lp - jnp.max(lp))
        learned_pi = e / jnp.sum(e)
        pi = _PRIOR_STRENGTH * learned_pi + (1.0 - _PRIOR_STRENGTH) * (
            counts / total)

        om = jax.lax.dot_general(
            pi.astype(jnp.bfloat16), mu_ref[...],
            (((1,), (0,)), ((), ())),
            preferred_element_type=jnp.float32)

        var = jnp.exp(lcv_ref[0, 0])
        inv_var = 1.0 / var
        within = (acc_ref[0] - 2.0 * acc_ref[1]
                  + jnp.sum(counts * m2)) * (inv_var / _N)
        between = (jnp.sum(pi * m2) - jnp.sum(om * om)) * inv_var
        out_ref[0, 0] = -(between / (within + _FISHER_EPS))


def kernel(z, y, log_cov, prior_logits):
    y3 = y.astype(jnp.int32).reshape(_GRID, 1, _BLK)
    mu = jnp.asarray(_MU_BF16)
    m2 = jnp.asarray(_M2)
    plp = jnp.full((1, _CPAD), -1e30, dtype=jnp.float32)
    plp = plp.at[0, :_C].set(prior_logits.astype(jnp.float32))
    lcv = log_cov.astype(jnp.float32).reshape(1, 1)

    out = pl.pallas_call(
        _fisher_kernel,
        grid=(_GRID,),
        in_specs=[
            pl.BlockSpec((_BLK, _D), lambda i: (i, 0)),
            pl.BlockSpec((1, 1, _BLK), lambda i: (i, 0, 0)),
            pl.BlockSpec((_CPAD, _D), lambda i: (0, 0)),
            pl.BlockSpec((1, _CPAD), lambda i: (0, 0)),
            pl.BlockSpec((1, _CPAD), lambda i: (0, 0)),
            pl.BlockSpec((1, 1), lambda i: (0, 0)),
        ],
        out_specs=pl.BlockSpec(memory_space=pltpu.SMEM),
        out_shape=jax.ShapeDtypeStruct((1, 1), jnp.float32),
        scratch_shapes=[
            pltpu.SMEM((2,), jnp.float32),
            pltpu.VMEM((1, _CPAD), jnp.float32),
        ],
        compiler_params=pltpu.CompilerParams(
            dimension_semantics=("arbitrary",)),
    )(z, y3, mu, m2, plp, lcv)
    return out[0, 0]


# speedup vs baseline: 84.0624x; 1.7072x over previous
import math

import numpy as np
import jax
import jax.numpy as jnp
from jax.experimental import pallas as pl
from jax.experimental.pallas import tpu as pltpu

_C = 1000
_D = 2048
_CPAD = 1024
_N = 16384
_BLK = 1024
_GRID = _N // _BLK
_FISHER_EPS = 1e-08
_PRIOR_STRENGTH = 0.5


def _build_mu() -> np.ndarray:
    k = np.arange(1, _C, dtype=np.float64)
    inv = 1.0 / np.sqrt(k * (k + 1.0))
    r = np.arange(_C, dtype=np.float64)[:, None]
    kk = k[None, :]
    basis = np.where(r < kk, inv, np.where(r == kk, -kk * inv, 0.0))
    pairwise = math.sqrt(2.0 * _C / (_C - 1))
    scale = 6.0 / pairwise
    mu = basis * (scale / math.sqrt(1.0 - 1.0 / _C))
    out = np.zeros((_CPAD, _D), dtype=np.float32)
    out[:_C, : _C - 1] = mu.astype(np.float32)
    return out


_MU_F32 = _build_mu()
_MU_BF16 = _MU_F32[:, :_CPAD].astype(jnp.bfloat16)
_M2 = (
    _MU_F32.astype(np.float64) ** 2
).sum(axis=1).astype(np.float32).reshape(1, _CPAD)


def _fisher_kernel(z_ref, y_ref, mu_ref, m2_ref, plp_ref, lcv_ref,
                   out_ref, acc_ref, counts_ref):
    i = pl.program_id(0)

    @pl.when(i == 0)
    def _init():
        acc_ref[0] = 0.0
        acc_ref[1] = 0.0
        counts_ref[...] = jnp.zeros_like(counts_ref)

    z = z_ref[...]
    y = y_ref[0, 0, :]
    yc = y.reshape(1, _BLK).T
    col = jax.lax.broadcasted_iota(jnp.int32, (_BLK, _CPAD), 1)
    onehot = (yc == col).astype(jnp.bfloat16)

    counts_ref[...] += jnp.sum(
        (yc == col).astype(jnp.float32), axis=0, keepdims=True)

    g = jax.lax.dot_general(
        onehot, mu_ref[...],
        (((1,), (0,)), ((), ())),
        preferred_element_type=jnp.float32)

    acc_ref[0] += jnp.sum(z * z)
    acc_ref[1] += jnp.sum(z[:, :_CPAD] * g)

    @pl.when(i == _GRID - 1)
    def _epilogue():
        counts = counts_ref[...]
        m2 = m2_ref[...]
        total = jnp.maximum(jnp.sum(counts), 1.0)

        lp = plp_ref[...]
        e = jnp.exp(lp - jnp.max(lp))
        learned_pi = e / jnp.sum(e)
        pi = _PRIOR_STRENGTH * learned_pi + (1.0 - _PRIOR_STRENGTH) * (
            counts / total)

        om = jax.lax.dot_general(
            pi.astype(jnp.bfloat16), mu_ref[...],
            (((1,), (0,)), ((), ())),
            preferred_element_type=jnp.float32)

        var = jnp.exp(lcv_ref[0, 0])
        inv_var = 1.0 / var
        within = (acc_ref[0] - 2.0 * acc_ref[1]
                  + jnp.sum(counts * m2)) * (inv_var / _N)
        between = (jnp.sum(pi * m2) - jnp.sum(om * om)) * inv_var
        out_ref[0, 0] = -(between / (within + _FISHER_EPS))


def kernel(z, y, log_cov, prior_logits):
    y3 = y.astype(jnp.int32).reshape(_GRID, 1, _BLK)
    mu = jnp.asarray(_MU_BF16)
    m2 = jnp.asarray(_M2)
    plp = jnp.full((1, _CPAD), -1e30, dtype=jnp.float32)
    plp = plp.at[0, :_C].set(prior_logits.astype(jnp.float32))
    lcv = log_cov.astype(jnp.float32).reshape(1, 1)

    out = pl.pallas_call(
        _fisher_kernel,
        grid=(_GRID,),
        in_specs=[
            pl.BlockSpec((_BLK, _D), lambda i: (i, 0)),
            pl.BlockSpec((1, 1, _BLK), lambda i: (i, 0, 0)),
            pl.BlockSpec((_CPAD, _CPAD), lambda i: (0, 0)),
            pl.BlockSpec((1, _CPAD), lambda i: (0, 0)),
            pl.BlockSpec((1, _CPAD), lambda i: (0, 0)),
            pl.BlockSpec((1, 1), lambda i: (0, 0)),
        ],
        out_specs=pl.BlockSpec(memory_space=pltpu.SMEM),
        out_shape=jax.ShapeDtypeStruct((1, 1), jnp.float32),
        scratch_shapes=[
            pltpu.SMEM((2,), jnp.float32),
            pltpu.VMEM((1, _CPAD), jnp.float32),
        ],
        compiler_params=pltpu.CompilerParams(
            dimension_semantics=("arbitrary",)),
    )(z, y3, mu, m2, plp, lcv)
    return out[0, 0]
